# Initial kernel scaffold; baseline (speedup 1.0000x reference)
import jax
import jax.numpy as jnp
from jax.experimental import pallas as pl


def kernel(x, Wr, br, W1, b1, W2, b2):
    raise NotImplementedError("write your pallas kernel here")


# speedup vs baseline: 1.2544x; 1.2544x over previous
import functools

import jax
import jax.numpy as jnp
from jax.experimental import pallas as pl
from jax.experimental.pallas import tpu as pltpu

E = 16
D = 1024
P = 4096
N = 4096
BN = 512
NB = N // BN


def _moe_dense_body(x_ref, Wr_ref, br_ref, W1_ref, b1_ref, W2_ref, b2_ref,
                    out_ref):
    e = pl.program_id(1)
    x = x_ref[...]
    xb = x.astype(jnp.bfloat16)
    scores = jax.lax.dot_general(
        xb, Wr_ref[...].astype(jnp.bfloat16), (((1,), (0,)), ((), ())),
        preferred_element_type=jnp.float32) + br_ref[...]
    lane = jax.lax.broadcasted_iota(jnp.int32, scores.shape, 1)
    v1 = jnp.max(scores, axis=1, keepdims=True)
    i1 = jnp.min(jnp.where(scores == v1, lane, E), axis=1, keepdims=True)
    s2 = jnp.where(lane == i1, -jnp.inf, scores)
    v2 = jnp.max(s2, axis=1, keepdims=True)
    i2 = jnp.min(jnp.where(s2 == v2, lane, E), axis=1, keepdims=True)
    w1 = jax.nn.sigmoid(v1 - v2)
    w2 = 1.0 - w1
    w_e = jnp.where(i1 == e, w1, jnp.where(i2 == e, w2, 0.0))

    h = jax.lax.dot_general(
        xb, W1_ref[0], (((1,), (0,)), ((), ())),
        preferred_element_type=jnp.float32) + b1_ref[0]
    h = jax.nn.gelu(h)
    y = jax.lax.dot_general(
        h.astype(jnp.bfloat16), W2_ref[0], (((1,), (0,)), ((), ())),
        preferred_element_type=jnp.float32) + b2_ref[0]
    contrib = w_e * y

    @pl.when(e == 0)
    def _init():
        out_ref[...] = contrib

    @pl.when(e > 0)
    def _acc():
        out_ref[...] += contrib


def kernel(x, Wr, br, W1, b1, W2, b2):
    W1b = W1.astype(jnp.bfloat16)
    W2b = W2.astype(jnp.bfloat16)
    br2 = br.reshape(1, E)
    b1r = b1.reshape(E, 1, P)
    b2r = b2.reshape(E, 1, D)
    out = pl.pallas_call(
        _moe_dense_body,
        grid=(NB, E),
        in_specs=[
            pl.BlockSpec((BN, D), lambda nb, e: (nb, 0)),
            pl.BlockSpec((D, E), lambda nb, e: (0, 0)),
            pl.BlockSpec((1, E), lambda nb, e: (0, 0)),
            pl.BlockSpec((1, D, P), lambda nb, e: (e, 0, 0)),
            pl.BlockSpec((1, 1, P), lambda nb, e: (e, 0, 0)),
            pl.BlockSpec((1, P, D), lambda nb, e: (e, 0, 0)),
            pl.BlockSpec((1, 1, D), lambda nb, e: (e, 0, 0)),
        ],
        out_specs=pl.BlockSpec((BN, D), lambda nb, e: (nb, 0)),
        out_shape=jax.ShapeDtypeStruct((N, D), jnp.float32),
    )(x, Wr, br2, W1b, b1r, W2b, b2r)
    return out


# speedup vs baseline: 1.9556x; 1.5589x over previous
import functools

import jax
import jax.numpy as jnp
from jax import lax
from jax.experimental import pallas as pl
from jax.experimental.pallas import tpu as pltpu
from jax.experimental.pallas import tpu_sc as plsc

E = 16
D = 1024
P = 4096
N = 4096
K = 2
M = 256
NT = (K * N) // M + E
S = NT * M
NW = 32
BR = 512


def _router_body(x_ref, Wr_ref, br_ref, idx_ref, w_ref):
    xb = x_ref[...].astype(jnp.bfloat16)
    scores = lax.dot_general(
        xb, Wr_ref[...].astype(jnp.bfloat16), (((1,), (0,)), ((), ())),
        preferred_element_type=jnp.float32) + br_ref[...]
    lane = lax.broadcasted_iota(jnp.int32, scores.shape, 1)
    v1 = jnp.max(scores, axis=1, keepdims=True)
    i1 = jnp.min(jnp.where(scores == v1, lane, E), axis=1, keepdims=True)
    s2 = jnp.where(lane == i1, -jnp.inf, scores)
    v2 = jnp.max(s2, axis=1, keepdims=True)
    i2 = jnp.min(jnp.where(s2 == v2, lane, E), axis=1, keepdims=True)
    w1 = jax.nn.sigmoid(v1 - v2)
    idx_ref[...] = jnp.concatenate([i1, i2], axis=1)
    w_ref[...] = jnp.concatenate([w1, 1.0 - w1], axis=1)


def _router(x, Wr, br):
    return pl.pallas_call(
        _router_body,
        grid=(N // BR,),
        in_specs=[pl.BlockSpec((BR, D), lambda i: (i, 0)),
                  pl.BlockSpec((D, E), lambda i: (0, 0)),
                  pl.BlockSpec((1, E), lambda i: (0, 0))],
        out_specs=[pl.BlockSpec((BR, K), lambda i: (i, 0)),
                   pl.BlockSpec((BR, K), lambda i: (i, 0))],
        out_shape=[jax.ShapeDtypeStruct((N, K), jnp.int32),
                   jax.ShapeDtypeStruct((N, K), jnp.float32)],
    )(x, Wr, br.reshape(1, E))


def _dispatch_meta(idx, w):
    e_flat = jnp.concatenate([idx[:, 0], idx[:, 1]])
    w_flat = jnp.concatenate([w[:, 0], w[:, 1]])
    t_flat = jnp.concatenate([jnp.arange(N, dtype=jnp.int32)] * 2)
    perm = jnp.argsort(e_flat, stable=True)
    se = e_flat[perm]
    st = t_flat[perm]
    sw = w_flat[perm]
    counts = jnp.zeros((E,), jnp.int32).at[e_flat].add(1)
    start = jnp.cumsum(counts) - counts
    pc = ((counts + M - 1) // M) * M
    pend = jnp.cumsum(pc)
    pstart = pend - pc
    j = jnp.arange(K * N, dtype=jnp.int32)
    slot_sorted = pstart[se] + (j - start[se])
    row_token = jnp.zeros((S,), jnp.int32).at[slot_sorted].set(st)
    w_slot = jnp.zeros((S,), jnp.float32).at[slot_sorted].set(sw)
    tile_expert = jnp.minimum(
        jnp.searchsorted(pend, jnp.arange(NT, dtype=jnp.int32) * M,
                         side='right'),
        E - 1).astype(jnp.int32)
    inv = jnp.zeros((K * N,), jnp.int32).at[perm].set(j)
    slot_flat = slot_sorted[inv]
    return row_token, w_slot, tile_expert, slot_flat


@functools.lru_cache(maxsize=None)
def _make_gather(table_rows, n_idx, chunk):
    rows_per_w = n_idx // NW
    n_chunks = rows_per_w // chunk
    mesh = plsc.VectorSubcoreMesh(core_axis_name="c", subcore_axis_name="s")

    @functools.partial(
        pl.kernel, mesh=mesh,
        out_type=jax.ShapeDtypeStruct((n_idx, D), jnp.float32),
        scratch_types=[pltpu.VMEM((chunk,), jnp.int32),
                       pltpu.VMEM((chunk, D), jnp.float32),
                       pltpu.SemaphoreType.DMA],
    )
    def gk(table_hbm, idx_hbm, out_hbm, idx_v, rows_v, sem):
        wid = lax.axis_index("s") * 2 + lax.axis_index("c")
        base = wid * rows_per_w

        def body(c, carry):
            off = base + c * chunk
            pltpu.sync_copy(idx_hbm.at[pl.ds(off, chunk)], idx_v)
            pltpu.async_copy(table_hbm.at[idx_v], rows_v, sem).wait()
            pltpu.sync_copy(rows_v, out_hbm.at[pl.ds(off, chunk)])
            return carry

        lax.fori_loop(0, n_chunks, body, 0)

    return gk


def _gather_rows(table, idx, chunk=64):
    return _make_gather(table.shape[0], idx.shape[0], chunk)(table, idx)


def _mm_body(te_ref, xg_ref, W1_ref, b1_ref, W2_ref, b2_ref, w_ref, out_ref):
    xb = xg_ref[...].astype(jnp.bfloat16)
    h = lax.dot_general(
        xb, W1_ref[0], (((1,), (0,)), ((), ())),
        preferred_element_type=jnp.float32) + b1_ref[0]
    h = jax.nn.gelu(h)
    y = lax.dot_general(
        h.astype(jnp.bfloat16), W2_ref[0], (((1,), (0,)), ((), ())),
        preferred_element_type=jnp.float32) + b2_ref[0]
    out_ref[...] = y * w_ref[0]


def _expert_matmul(tile_expert, xg, W1b, b1r, W2b, b2r, w3):
    grid_spec = pltpu.PrefetchScalarGridSpec(
        num_scalar_prefetch=1,
        grid=(NT,),
        in_specs=[
            pl.BlockSpec((M, D), lambda t, te: (t, 0)),
            pl.BlockSpec((1, D, P), lambda t, te: (te[t], 0, 0)),
            pl.BlockSpec((1, 1, P), lambda t, te: (te[t], 0, 0)),
            pl.BlockSpec((1, P, D), lambda t, te: (te[t], 0, 0)),
            pl.BlockSpec((1, 1, D), lambda t, te: (te[t], 0, 0)),
            pl.BlockSpec((1, M, 1), lambda t, te: (t, 0, 0)),
        ],
        out_specs=pl.BlockSpec((M, D), lambda t, te: (t, 0)),
    )
    return pl.pallas_call(
        _mm_body,
        grid_spec=grid_spec,
        out_shape=jax.ShapeDtypeStruct((S, D), jnp.float32),
    )(tile_expert, xg, W1b, b1r, W2b, b2r, w3)


def _add_body(a_ref, b_ref, o_ref):
    o_ref[...] = a_ref[...] + b_ref[...]


def _combine(yg):
    nb = N // BR
    return pl.pallas_call(
        _add_body,
        grid=(nb,),
        in_specs=[pl.BlockSpec((BR, D), lambda i: (i, 0)),
                  pl.BlockSpec((BR, D), lambda i, _nb=nb: (i + _nb, 0))],
        out_specs=pl.BlockSpec((BR, D), lambda i: (i, 0)),
        out_shape=jax.ShapeDtypeStruct((N, D), jnp.float32),
    )(yg, yg)


def kernel(x, Wr, br, W1, b1, W2, b2):
    idx, w = _router(x, Wr, br)
    row_token, w_slot, tile_expert, slot_flat = _dispatch_meta(idx, w)
    xg = _gather_rows(x, row_token)
    ys = _expert_matmul(
        tile_expert, xg,
        W1.astype(jnp.bfloat16), b1.reshape(E, 1, P),
        W2.astype(jnp.bfloat16), b2.reshape(E, 1, D),
        w_slot.reshape(NT, M, 1))
    yg = _gather_rows(ys, slot_flat)
    return _combine(yg)


# speedup vs baseline: 2.4839x; 1.2702x over previous
import functools

import jax
import jax.numpy as jnp
from jax import lax
from jax.experimental import pallas as pl
from jax.experimental.pallas import tpu as pltpu
from jax.experimental.pallas import tpu_sc as plsc

E = 16
D = 1024
P = 4096
N = 4096
K = 2
M = 256
NT = (K * N) // M + E
S = NT * M
NW = 32
BR = 512


def _router_body(x_ref, Wr_ref, br_ref, idx_ref, w_ref):
    xb = x_ref[...].astype(jnp.bfloat16)
    scores = lax.dot_general(
        xb, Wr_ref[...].astype(jnp.bfloat16), (((1,), (0,)), ((), ())),
        preferred_element_type=jnp.float32) + br_ref[...]
    lane = lax.broadcasted_iota(jnp.int32, scores.shape, 1)
    v1 = jnp.max(scores, axis=1, keepdims=True)
    i1 = jnp.min(jnp.where(scores == v1, lane, E), axis=1, keepdims=True)
    s2 = jnp.where(lane == i1, -jnp.inf, scores)
    v2 = jnp.max(s2, axis=1, keepdims=True)
    i2 = jnp.min(jnp.where(s2 == v2, lane, E), axis=1, keepdims=True)
    w1 = jax.nn.sigmoid(v1 - v2)
    idx_ref[...] = jnp.concatenate([i1, i2], axis=1)
    w_ref[...] = jnp.concatenate([w1, 1.0 - w1], axis=1)


def _router(x, Wr, br):
    return pl.pallas_call(
        _router_body,
        grid=(N // BR,),
        in_specs=[pl.BlockSpec((BR, D), lambda i: (i, 0)),
                  pl.BlockSpec((D, E), lambda i: (0, 0)),
                  pl.BlockSpec((1, E), lambda i: (0, 0))],
        out_specs=[pl.BlockSpec((BR, K), lambda i: (i, 0)),
                   pl.BlockSpec((BR, K), lambda i: (i, 0))],
        out_shape=[jax.ShapeDtypeStruct((N, K), jnp.int32),
                   jax.ShapeDtypeStruct((N, K), jnp.float32)],
    )(x, Wr, br.reshape(1, E))


def _dispatch_meta(idx, w):
    e_flat = jnp.concatenate([idx[:, 0], idx[:, 1]])
    w_flat = jnp.concatenate([w[:, 0], w[:, 1]])
    t_flat = jnp.concatenate([jnp.arange(N, dtype=jnp.int32)] * 2)
    perm = jnp.argsort(e_flat, stable=True)
    se = e_flat[perm]
    st = t_flat[perm]
    sw = w_flat[perm]
    counts = jnp.zeros((E,), jnp.int32).at[e_flat].add(1)
    start = jnp.cumsum(counts) - counts
    pc = ((counts + M - 1) // M) * M
    pend = jnp.cumsum(pc)
    pstart = pend - pc
    j = jnp.arange(K * N, dtype=jnp.int32)
    slot_sorted = pstart[se] + (j - start[se])
    row_token = (jnp.arange(S, dtype=jnp.int32) % N).at[slot_sorted].set(st)
    w_slot = jnp.zeros((S,), jnp.float32).at[slot_sorted].set(sw)
    tile_expert = jnp.minimum(
        jnp.searchsorted(pend, jnp.arange(NT, dtype=jnp.int32) * M,
                         side='right'),
        E - 1).astype(jnp.int32)
    inv = jnp.zeros((K * N,), jnp.int32).at[perm].set(j)
    slot_flat = slot_sorted[inv]
    return row_token, w_slot, tile_expert, slot_flat


@functools.lru_cache(maxsize=None)
def _make_gather(table_rows, n_idx, chunk):
    rows_per_w = n_idx // NW
    n_chunks = rows_per_w // chunk
    mesh = plsc.VectorSubcoreMesh(core_axis_name="c", subcore_axis_name="s")

    @functools.partial(
        pl.kernel, mesh=mesh,
        out_type=jax.ShapeDtypeStruct((n_idx, D), jnp.float32),
        scratch_types=[pltpu.VMEM((chunk,), jnp.int32),
                       pltpu.VMEM((chunk, D), jnp.float32),
                       pltpu.SemaphoreType.DMA],
    )
    def gk(table_hbm, idx_hbm, out_hbm, idx_v, rows_v, sem):
        wid = lax.axis_index("s") * 2 + lax.axis_index("c")
        base = wid * rows_per_w

        def body(c, carry):
            off = base + c * chunk
            pltpu.sync_copy(idx_hbm.at[pl.ds(off, chunk)], idx_v)
            pltpu.async_copy(table_hbm.at[idx_v], rows_v, sem).wait()
            pltpu.sync_copy(rows_v, out_hbm.at[pl.ds(off, chunk)])
            return carry

        lax.fori_loop(0, n_chunks, body, 0)

    return gk


def _gather_rows(table, idx, chunk=64):
    return _make_gather(table.shape[0], idx.shape[0], chunk)(table, idx)


def _mm_body(te_ref, xg_ref, W1_ref, b1_ref, W2_ref, b2_ref, w_ref, out_ref):
    xb = xg_ref[...].astype(jnp.bfloat16)
    h = lax.dot_general(
        xb, W1_ref[0], (((1,), (0,)), ((), ())),
        preferred_element_type=jnp.float32) + b1_ref[0]
    h = jax.nn.gelu(h)
    y = lax.dot_general(
        h.astype(jnp.bfloat16), W2_ref[0], (((1,), (0,)), ((), ())),
        preferred_element_type=jnp.float32) + b2_ref[0]
    out_ref[...] = y * w_ref[0]


def _expert_matmul(tile_expert, xg, W1b, b1r, W2b, b2r, w3):
    grid_spec = pltpu.PrefetchScalarGridSpec(
        num_scalar_prefetch=1,
        grid=(NT,),
        in_specs=[
            pl.BlockSpec((M, D), lambda t, te: (t, 0)),
            pl.BlockSpec((1, D, P), lambda t, te: (te[t], 0, 0)),
            pl.BlockSpec((1, 1, P), lambda t, te: (te[t], 0, 0)),
            pl.BlockSpec((1, P, D), lambda t, te: (te[t], 0, 0)),
            pl.BlockSpec((1, 1, D), lambda t, te: (te[t], 0, 0)),
            pl.BlockSpec((1, M, 1), lambda t, te: (t, 0, 0)),
        ],
        out_specs=pl.BlockSpec((M, D), lambda t, te: (t, 0)),
    )
    return pl.pallas_call(
        _mm_body,
        grid_spec=grid_spec,
        out_shape=jax.ShapeDtypeStruct((S, D), jnp.float32),
    )(tile_expert, xg, W1b, b1r, W2b, b2r, w3)


def _add_body(a_ref, b_ref, o_ref):
    o_ref[...] = a_ref[...] + b_ref[...]


def _combine(yg):
    nb = N // BR
    return pl.pallas_call(
        _add_body,
        grid=(nb,),
        in_specs=[pl.BlockSpec((BR, D), lambda i: (i, 0)),
                  pl.BlockSpec((BR, D), lambda i, _nb=nb: (i + _nb, 0))],
        out_specs=pl.BlockSpec((BR, D), lambda i: (i, 0)),
        out_shape=jax.ShapeDtypeStruct((N, D), jnp.float32),
    )(yg, yg)


def kernel(x, Wr, br, W1, b1, W2, b2):
    idx, w = _router(x, Wr, br)
    row_token, w_slot, tile_expert, slot_flat = _dispatch_meta(idx, w)
    xg = _gather_rows(x, row_token)
    ys = _expert_matmul(
        tile_expert, xg,
        W1.astype(jnp.bfloat16), b1.reshape(E, 1, P),
        W2.astype(jnp.bfloat16), b2.reshape(E, 1, D),
        w_slot.reshape(NT, M, 1))
    yg = _gather_rows(ys, slot_flat)
    return _combine(yg)


# speedup vs baseline: 2.9599x; 1.1917x over previous
import functools

import jax
import jax.numpy as jnp
from jax import lax
from jax.experimental import pallas as pl
from jax.experimental.pallas import tpu as pltpu
from jax.experimental.pallas import tpu_sc as plsc

E = 16
D = 1024
P = 4096
N = 4096
K = 2
M = 256
NT = (K * N) // M + E
S = NT * M
NW = 32
BR = 512


def _router_body(x_ref, Wr_ref, br_ref, idx_ref, w_ref):
    xb = x_ref[...].astype(jnp.bfloat16)
    scores = lax.dot_general(
        xb, Wr_ref[...].astype(jnp.bfloat16), (((1,), (0,)), ((), ())),
        preferred_element_type=jnp.float32) + br_ref[...]
    lane = lax.broadcasted_iota(jnp.int32, scores.shape, 1)
    v1 = jnp.max(scores, axis=1, keepdims=True)
    i1 = jnp.min(jnp.where(scores == v1, lane, E), axis=1, keepdims=True)
    s2 = jnp.where(lane == i1, -jnp.inf, scores)
    v2 = jnp.max(s2, axis=1, keepdims=True)
    i2 = jnp.min(jnp.where(s2 == v2, lane, E), axis=1, keepdims=True)
    w1 = jax.nn.sigmoid(v1 - v2)
    idx_ref[...] = jnp.concatenate([i1, i2], axis=1)
    w_ref[...] = jnp.concatenate([w1, 1.0 - w1], axis=1)


def _router(x, Wr, br):
    return pl.pallas_call(
        _router_body,
        grid=(N // BR,),
        in_specs=[pl.BlockSpec((BR, D), lambda i: (i, 0)),
                  pl.BlockSpec((D, E), lambda i: (0, 0)),
                  pl.BlockSpec((1, E), lambda i: (0, 0))],
        out_specs=[pl.BlockSpec((BR, K), lambda i: (i, 0)),
                   pl.BlockSpec((BR, K), lambda i: (i, 0))],
        out_shape=[jax.ShapeDtypeStruct((N, K), jnp.int32),
                   jax.ShapeDtypeStruct((N, K), jnp.float32)],
    )(x, Wr, br.reshape(1, E))


def _dispatch_meta(idx, w):
    e_flat = jnp.concatenate([idx[:, 0], idx[:, 1]])
    w_flat = jnp.concatenate([w[:, 0], w[:, 1]])
    t_flat = jnp.concatenate([jnp.arange(N, dtype=jnp.int32)] * 2)
    perm = jnp.argsort(e_flat, stable=True)
    se = e_flat[perm]
    st = t_flat[perm]
    sw = w_flat[perm]
    counts = jnp.zeros((E,), jnp.int32).at[e_flat].add(1)
    start = jnp.cumsum(counts) - counts
    pc = ((counts + M - 1) // M) * M
    pend = jnp.cumsum(pc)
    pstart = pend - pc
    j = jnp.arange(K * N, dtype=jnp.int32)
    slot_sorted = pstart[se] + (j - start[se])
    row_token = (jnp.arange(S, dtype=jnp.int32) % N).at[slot_sorted].set(st)
    w_slot = jnp.zeros((S,), jnp.float32).at[slot_sorted].set(sw)
    tile_expert = jnp.minimum(
        jnp.searchsorted(pend, jnp.arange(NT, dtype=jnp.int32) * M,
                         side='right'),
        E - 1).astype(jnp.int32)
    inv = jnp.zeros((K * N,), jnp.int32).at[perm].set(j)
    slot_flat = slot_sorted[inv]
    return row_token, w_slot, tile_expert, slot_flat


@functools.lru_cache(maxsize=None)
def _make_gather(table_rows, n_idx, chunk):
    rows_per_w = n_idx // NW
    n_chunks = rows_per_w // chunk
    mesh = plsc.VectorSubcoreMesh(core_axis_name="c", subcore_axis_name="s")

    @functools.partial(
        pl.kernel, mesh=mesh,
        out_type=jax.ShapeDtypeStruct((n_idx, D), jnp.float32),
        scratch_types=[pltpu.VMEM((chunk,), jnp.int32),
                       pltpu.VMEM((chunk, D), jnp.float32),
                       pltpu.SemaphoreType.DMA],
    )
    def gk(table_hbm, idx_hbm, out_hbm, idx_v, rows_v, sem):
        wid = lax.axis_index("s") * 2 + lax.axis_index("c")
        base = wid * rows_per_w

        def body(c, carry):
            off = base + c * chunk
            pltpu.sync_copy(idx_hbm.at[pl.ds(off, chunk)], idx_v)
            pltpu.async_copy(table_hbm.at[idx_v], rows_v, sem).wait()
            pltpu.sync_copy(rows_v, out_hbm.at[pl.ds(off, chunk)])
            return carry

        lax.fori_loop(0, n_chunks, body, 0)

    return gk


def _gather_rows(table, idx, chunk=64):
    return _make_gather(table.shape[0], idx.shape[0], chunk)(table, idx)


def _mm_body(te_ref, xg_ref, b1_ref, b2_ref, w_ref, W1_hbm, W2_hbm, out_ref,
             w1v, w2v, sem1, sem2):
    t = pl.program_id(0)
    e = te_ref[t]
    prev_e = te_ref[jnp.maximum(t - 1, 0)]

    @pl.when(jnp.logical_or(t == 0, e != prev_e))
    def _fetch():
        c1 = pltpu.make_async_copy(W1_hbm.at[e], w1v, sem1)
        c2 = pltpu.make_async_copy(W2_hbm.at[e], w2v, sem2)
        c1.start()
        c2.start()
        c1.wait()
        c2.wait()

    h = lax.dot_general(
        xg_ref[...], w1v[...], (((1,), (0,)), ((), ())),
        preferred_element_type=jnp.float32) + b1_ref[0]
    h = jax.nn.gelu(h)
    y = lax.dot_general(
        h, w2v[...], (((1,), (0,)), ((), ())),
        preferred_element_type=jnp.float32) + b2_ref[0]
    out_ref[...] = y * w_ref[0]


def _expert_matmul(tile_expert, xg, W1, b1r, W2, b2r, w3):
    grid_spec = pltpu.PrefetchScalarGridSpec(
        num_scalar_prefetch=1,
        grid=(NT,),
        in_specs=[
            pl.BlockSpec((M, D), lambda t, te: (t, 0)),
            pl.BlockSpec((1, 1, P), lambda t, te: (te[t], 0, 0)),
            pl.BlockSpec((1, 1, D), lambda t, te: (te[t], 0, 0)),
            pl.BlockSpec((1, M, 1), lambda t, te: (t, 0, 0)),
            pl.BlockSpec(memory_space=pl.ANY),
            pl.BlockSpec(memory_space=pl.ANY),
        ],
        out_specs=pl.BlockSpec((M, D), lambda t, te: (t, 0)),
        scratch_shapes=[
            pltpu.VMEM((D, P), jnp.float32),
            pltpu.VMEM((P, D), jnp.float32),
            pltpu.SemaphoreType.DMA,
            pltpu.SemaphoreType.DMA,
        ],
    )
    return pl.pallas_call(
        _mm_body,
        grid_spec=grid_spec,
        out_shape=jax.ShapeDtypeStruct((S, D), jnp.float32),
    )(tile_expert, xg, b1r, b2r, w3, W1, W2)


def _add_body(a_ref, b_ref, o_ref):
    o_ref[...] = a_ref[...] + b_ref[...]


def _combine(yg):
    nb = N // BR
    return pl.pallas_call(
        _add_body,
        grid=(nb,),
        in_specs=[pl.BlockSpec((BR, D), lambda i: (i, 0)),
                  pl.BlockSpec((BR, D), lambda i, _nb=nb: (i + _nb, 0))],
        out_specs=pl.BlockSpec((BR, D), lambda i: (i, 0)),
        out_shape=jax.ShapeDtypeStruct((N, D), jnp.float32),
    )(yg, yg)


def kernel(x, Wr, br, W1, b1, W2, b2):
    idx, w = _router(x, Wr, br)
    row_token, w_slot, tile_expert, slot_flat = _dispatch_meta(idx, w)
    xg = _gather_rows(x, row_token)
    ys = _expert_matmul(
        tile_expert, xg,
        W1, b1.reshape(E, 1, P),
        W2, b2.reshape(E, 1, D),
        w_slot.reshape(NT, M, 1))
    yg = _gather_rows(ys, slot_flat)
    return _combine(yg)
